# baseline (speedup 1.0000x reference)
import functools

import jax
import jax.numpy as jnp
from jax import lax
from jax.experimental import pallas as pl
from jax.experimental.pallas import tpu as pltpu
from jax.experimental.pallas import tpu_sc as plsc

LANES = 16
NC, NS = 2, 16
NW = NC * NS


@functools.lru_cache(maxsize=None)
def _build_sc_kernel(N, L, D, V):
    TG = LANES
    RG = L * TG
    HL = D * TG
    n_per_w = N // NW
    n_groups = n_per_w // TG
    n_pairs = n_groups // 2
    assert n_per_w * NW == N and n_pairs * 2 * TG == n_per_w
    assert L == LANES + 1

    mesh = plsc.VectorSubcoreMesh(core_axis_name="c", subcore_axis_name="s")

    @functools.partial(
        pl.kernel,
        mesh=mesh,
        compiler_params=pltpu.CompilerParams(needs_layout_passes=False),
        out_type=jax.ShapeDtypeStruct((N,), jnp.float32),
        scratch_types=[
            pltpu.VMEM((2 * TG, L), jnp.int32),
            pltpu.VMEM((2 * TG, L), jnp.float32),
            pltpu.VMEM((2 * HL,), jnp.float32),
            pltpu.VMEM((2 * RG, D), jnp.float32),
            pltpu.VMEM((2 * TG,), jnp.int32),
            pltpu.VMEM((2 * TG,), jnp.float32),
            pltpu.SemaphoreType.DMA,
            pltpu.SemaphoreType.DMA,
            pltpu.SemaphoreType.DMA,
        ],
    )
    def body(xn, zn, hf, w, out, xs, zs, hs, rows, vidx, outv,
             sem_s, sem_r0, sem_r1):
        wid = lax.axis_index("c") * NS + lax.axis_index("s")
        lane = lax.iota(jnp.int32, LANES)
        base = wid * n_per_w

        def stage_copies(slot, tb):
            return (
                pltpu.make_async_copy(xn.at[pl.ds(tb, TG), :],
                                      xs.at[pl.ds(slot * TG, TG), :], sem_s),
                pltpu.make_async_copy(zn.at[pl.ds(tb, TG), :],
                                      zs.at[pl.ds(slot * TG, TG), :], sem_s),
                pltpu.make_async_copy(hf.at[pl.ds(tb * D, HL)],
                                      hs.at[pl.ds(slot * HL, HL)], sem_s),
            )

        def fire_stage(slot, tb):
            for c in stage_copies(slot, tb):
                c.start()

        def drain_stage(slot, tb):
            for c in stage_copies(slot, tb):
                c.wait()

        def row_copies(slot, tb):
            sem = sem_r0 if slot == 0 else sem_r1
            cps = [
                pltpu.make_async_copy(
                    w.at[xs.at[slot * TG + t, pl.ds(0, LANES)]],
                    rows.at[pl.ds(slot * RG + t * LANES, LANES)],
                    sem,
                )
                for t in range(TG)
            ]
            cps.append(
                pltpu.make_async_copy(
                    w.at[vidx.at[pl.ds(slot * TG, TG)]],
                    rows.at[pl.ds(slot * RG + TG * LANES, TG)],
                    sem,
                )
            )
            return cps

        def fire_rows(slot, tb):
            x17 = plsc.load_gather(
                xs, [slot * TG + lane, jnp.full((LANES,), L - 1, jnp.int32)])
            vidx[pl.ds(slot * TG, TG)] = x17
            for c in row_copies(slot, tb):
                c.start()

        def drain_rows(slot, tb):
            for c in row_copies(slot, tb):
                c.wait()

        def compute(slot, out_half):
            rbase = slot * RG
            hbase = slot * HL
            perms = [jnp.bitwise_xor(lane, k) for k in (1, 2, 4, 8)]

            vb = rbase + TG * LANES

            def tstep(tp, carry):
                acc_out, dots17 = carry
                hvs = [hs[pl.ds(hbase + tp * D + j * LANES, LANES)]
                       for j in range(D // LANES)]
                rb = rbase + tp * LANES
                dots = jnp.zeros((LANES,), jnp.float32)
                for s in range(LANES):
                    acc = rows[rb + s, pl.ds(0, LANES)] * hvs[0]
                    for j in range(1, D // LANES):
                        acc = acc + rows[rb + s, pl.ds(j * LANES, LANES)] * hvs[j]
                    for p in perms:
                        acc = acc + jnp.take_along_axis(
                            acc, p, axis=0, mode="promise_in_bounds")
                    dots = jnp.where(lane == s, acc, dots)
                acc = rows[vb + tp, pl.ds(0, LANES)] * hvs[0]
                for j in range(1, D // LANES):
                    acc = acc + rows[vb + tp, pl.ds(j * LANES, LANES)] * hvs[j]
                for p in perms:
                    acc = acc + jnp.take_along_axis(
                        acc, p, axis=0, mode="promise_in_bounds")
                dots17 = jnp.where(lane == tp, acc, dots17)
                zrow = zs[slot * TG + tp, pl.ds(0, LANES)]
                xrow = xs[slot * TG + tp, pl.ds(0, LANES)]
                y = 1.0 / (1.0 + jnp.exp(-dots * zrow))
                y = jnp.where(xrow != 0, y, 1.0)
                for p in perms:
                    y = y * jnp.take_along_axis(
                        y, p, axis=0, mode="promise_in_bounds")
                return jnp.where(lane == tp, y, acc_out), dots17

            prod, dots17 = lax.fori_loop(
                0, TG, tstep,
                (jnp.ones((LANES,), jnp.float32),
                 jnp.zeros((LANES,), jnp.float32)))

            z17 = plsc.load_gather(
                zs, [slot * TG + lane, jnp.full((LANES,), L - 1, jnp.int32)])
            x17 = vidx[pl.ds(slot * TG, TG)]
            y17 = 1.0 / (1.0 + jnp.exp(-dots17 * z17))
            y17 = jnp.where(x17 != 0, y17, 1.0)
            outv[pl.ds(out_half * TG, TG)] = prod * y17

        fire_stage(0, base)
        drain_stage(0, base)
        fire_rows(0, base)
        fire_stage(1, base + TG)

        def pair(g2, carry):
            tb0 = base + g2 * (2 * TG)
            tb1 = tb0 + TG
            tb2 = tb0 + 2 * TG
            not_last = g2 < n_pairs - 1

            drain_stage(1, tb1)
            fire_rows(1, tb1)
            drain_rows(0, tb0)
            compute(0, 0)

            @pl.when(not_last)
            def _():
                fire_stage(0, tb2)

            @pl.when(not_last)
            def _():
                drain_stage(0, tb2)
                fire_rows(0, tb2)

            drain_rows(1, tb1)
            compute(1, 1)

            @pl.when(not_last)
            def _():
                fire_stage(1, tb2 + TG)

            pltpu.sync_copy(outv, out.at[pl.ds(tb0, 2 * TG)])
            return carry

        lax.fori_loop(0, n_pairs, pair, 0)

    return body


def kernel(x, z, h, w):
    B, T, L = x.shape
    D = h.shape[-1]
    N = B * T
    xn = x.reshape(N, L).astype(jnp.int32)
    zn = z.reshape(N, L).astype(jnp.float32)
    hf = h.reshape(-1).astype(jnp.float32)
    out = _build_sc_kernel(N, L, D, w.shape[0])(xn, zn, hf, w.astype(jnp.float32))
    return out.reshape(B, T)


# speedup vs baseline: 1.1028x; 1.1028x over previous
import functools

import jax
import jax.numpy as jnp
from jax import lax
from jax.experimental import pallas as pl
from jax.experimental.pallas import tpu as pltpu
from jax.experimental.pallas import tpu_sc as plsc

LANES = 16
NC, NS = 2, 16
NW = NC * NS


@functools.lru_cache(maxsize=None)
def _build_sc_kernel(N, L, D, V):
    TG = LANES
    RG = L * TG
    HL = D * TG
    n_per_w = N // NW
    n_groups = n_per_w // TG
    n_pairs = n_groups // 2
    assert n_per_w * NW == N and n_pairs * 2 * TG == n_per_w
    assert L == LANES + 1

    mesh = plsc.VectorSubcoreMesh(core_axis_name="c", subcore_axis_name="s")

    @functools.partial(
        pl.kernel,
        mesh=mesh,
        compiler_params=pltpu.CompilerParams(needs_layout_passes=False),
        out_type=jax.ShapeDtypeStruct((N,), jnp.float32),
        scratch_types=[
            pltpu.VMEM((2 * TG, L), jnp.int32),
            pltpu.VMEM((2 * TG, L), jnp.float32),
            pltpu.VMEM((2 * HL,), jnp.float32),
            pltpu.VMEM((2 * RG, D), jnp.float32),
            pltpu.VMEM((2 * TG,), jnp.int32),
            pltpu.VMEM((2 * TG,), jnp.float32),
            pltpu.SemaphoreType.DMA,
            pltpu.SemaphoreType.DMA,
            pltpu.SemaphoreType.DMA,
        ],
    )
    def body(xn, zn, hf, w, out, xs, zs, hs, rows, vidx, outv,
             sem_s, sem_r0, sem_r1):
        wid = lax.axis_index("c") * NS + lax.axis_index("s")
        lane = lax.iota(jnp.int32, LANES)
        base = wid * n_per_w

        def stage_copies(slot, tb):
            return (
                pltpu.make_async_copy(xn.at[pl.ds(tb, TG), :],
                                      xs.at[pl.ds(slot * TG, TG), :], sem_s),
                pltpu.make_async_copy(zn.at[pl.ds(tb, TG), :],
                                      zs.at[pl.ds(slot * TG, TG), :], sem_s),
                pltpu.make_async_copy(hf.at[pl.ds(tb * D, HL)],
                                      hs.at[pl.ds(slot * HL, HL)], sem_s),
            )

        def fire_stage(slot, tb):
            for c in stage_copies(slot, tb):
                c.start()

        def drain_stage(slot, tb):
            for c in stage_copies(slot, tb):
                c.wait()

        def row_copies(slot, tb):
            sem = sem_r0 if slot == 0 else sem_r1
            cps = [
                pltpu.make_async_copy(
                    w.at[xs.at[slot * TG + t, pl.ds(0, LANES)]],
                    rows.at[pl.ds(slot * RG + t * LANES, LANES)],
                    sem,
                )
                for t in range(TG)
            ]
            cps.append(
                pltpu.make_async_copy(
                    w.at[vidx.at[pl.ds(slot * TG, TG)]],
                    rows.at[pl.ds(slot * RG + TG * LANES, TG)],
                    sem,
                )
            )
            return cps

        def fire_rows(slot, tb):
            x17 = plsc.load_gather(
                xs, [slot * TG + lane, jnp.full((LANES,), L - 1, jnp.int32)])
            vidx[pl.ds(slot * TG, TG)] = x17
            for c in row_copies(slot, tb):
                c.start()

        def drain_rows(slot, tb):
            for c in row_copies(slot, tb):
                c.wait()

        def compute(slot, out_half):
            rbase = slot * RG
            hbase = slot * HL
            perms = [jnp.bitwise_xor(lane, k) for k in (1, 2, 4, 8)]

            def tstep(tp, acc_out):
                hvs = [hs[pl.ds(hbase + tp * D + j * LANES, LANES)]
                       for j in range(D // LANES)]
                rb = rbase + tp * LANES
                dots = jnp.zeros((LANES,), jnp.float32)
                for s in range(LANES):
                    acc = rows[rb + s, pl.ds(0, LANES)] * hvs[0]
                    for j in range(1, D // LANES):
                        acc = acc + rows[rb + s, pl.ds(j * LANES, LANES)] * hvs[j]
                    for p in perms:
                        acc = acc + jnp.take_along_axis(
                            acc, p, axis=0, mode="promise_in_bounds")
                    dots = jnp.where(lane == s, acc, dots)
                zrow = zs[slot * TG + tp, pl.ds(0, LANES)]
                xrow = xs[slot * TG + tp, pl.ds(0, LANES)]
                y = 1.0 / (1.0 + jnp.exp(-dots * zrow))
                y = jnp.where(xrow != 0, y, 1.0)
                for p in perms:
                    y = y * jnp.take_along_axis(
                        y, p, axis=0, mode="promise_in_bounds")
                return jnp.where(lane == tp, y, acc_out)

            prod = lax.fori_loop(0, TG, tstep, jnp.ones((LANES,), jnp.float32))

            vb = rbase + TG * LANES
            dots17 = jnp.zeros((LANES,), jnp.float32)
            for t in range(TG):
                acc = rows[vb + t, pl.ds(0, LANES)] * hs[pl.ds(hbase + t * D, LANES)]
                for j in range(1, D // LANES):
                    acc = acc + (rows[vb + t, pl.ds(j * LANES, LANES)]
                                 * hs[pl.ds(hbase + t * D + j * LANES, LANES)])
                for p in perms:
                    acc = acc + jnp.take_along_axis(
                        acc, p, axis=0, mode="promise_in_bounds")
                dots17 = jnp.where(lane == t, acc, dots17)
            z17 = plsc.load_gather(
                zs, [slot * TG + lane, jnp.full((LANES,), L - 1, jnp.int32)])
            x17 = vidx[pl.ds(slot * TG, TG)]
            y17 = 1.0 / (1.0 + jnp.exp(-dots17 * z17))
            y17 = jnp.where(x17 != 0, y17, 1.0)
            outv[pl.ds(out_half * TG, TG)] = prod * y17

        fire_stage(0, base)
        drain_stage(0, base)
        fire_rows(0, base)
        fire_stage(1, base + TG)

        def pair(g2, carry):
            tb0 = base + g2 * (2 * TG)
            tb1 = tb0 + TG
            tb2 = tb0 + 2 * TG
            not_last = g2 < n_pairs - 1

            drain_stage(1, tb1)
            fire_rows(1, tb1)
            drain_rows(0, tb0)
            compute(0, 0)

            @pl.when(not_last)
            def _():
                fire_stage(0, tb2)

            @pl.when(not_last)
            def _():
                drain_stage(0, tb2)
                fire_rows(0, tb2)

            drain_rows(1, tb1)
            compute(1, 1)

            @pl.when(not_last)
            def _():
                fire_stage(1, tb2 + TG)

            pltpu.sync_copy(outv, out.at[pl.ds(tb0, 2 * TG)])
            return carry

        lax.fori_loop(0, n_pairs, pair, 0)

    return body


def kernel(x, z, h, w):
    B, T, L = x.shape
    D = h.shape[-1]
    N = B * T
    xn = x.reshape(N, L).astype(jnp.int32)
    zn = z.reshape(N, L).astype(jnp.float32)
    hf = h.reshape(-1).astype(jnp.float32)
    out = _build_sc_kernel(N, L, D, w.shape[0])(xn, zn, hf, w.astype(jnp.float32))
    return out.reshape(B, T)
